# Initial kernel scaffold; baseline (speedup 1.0000x reference)
import jax
import jax.numpy as jnp
from jax.experimental import pallas as pl


def kernel(news_title, news_content, tweet_content, tweet_profile, user_profile, user_description, source_description, W_nt_t, b_nt_t, W_nt_c, b_nt_c, W_tw_c, b_tw_c, W_tw_p, b_tw_p, W_us_p, b_us_p, W_us_d, b_us_d, W_sr_d, b_sr_d, W_rte_nt, b_rte_nt, W_rte_tu, b_rte_tu, W_rte_uu, b_rte_uu, W_rte_ns, b_rte_ns, token, pos_emb, W_qkv, W_out, b_out, W_cls, b_cls, t_news, t_tweet, t_user, t_source, ei_nt, ei_tu, ei_uu, ei_ns):
    raise NotImplementedError("write your pallas kernel here")


# speedup vs baseline: 9.1142x; 9.1142x over previous
import functools

import jax
import jax.numpy as jnp
from jax.experimental import pallas as pl

_B = 4096
_H = 128
_HEADS = 4
_SCALE = (_H // _HEADS) ** -0.5
_TILE = 1024


def _fused_kernel(title_ref, content_ref, W1_ref, b1_ref, W2_ref, b2_ref,
                  tok_ref, pos_ref, Wqkv_ref, Wout_ref, bout_ref,
                  Wcls_ref, bcls_ref, out_ref):
    f32 = jnp.float32
    title = title_ref[...]
    content = content_ref[...]
    a = jax.lax.dot(title, W1_ref[...], preferred_element_type=f32) + b1_ref[...]
    b = jax.lax.dot(content, W2_ref[...], preferred_element_type=f32) + b2_ref[...]
    x = jnp.maximum(jnp.concatenate([a, b], axis=1), 0.0)

    pos0 = pos_ref[0:1, :]
    pos1 = pos_ref[1:2, :]
    s0 = tok_ref[...] + pos0
    s1 = x + pos1

    Wqkv = Wqkv_ref[...]
    qkv0 = jax.lax.dot(s0, Wqkv, preferred_element_type=f32)
    q0 = qkv0[:, 0:512]
    k0 = qkv0[:, 512:1024]
    v0 = qkv0[:, 1024:1536]
    k1 = jax.lax.dot(s1, Wqkv[:, 512:1024], preferred_element_type=f32)
    v1 = jax.lax.dot(s1, Wqkv[:, 1024:1536], preferred_element_type=f32)

    o_parts = []
    for h in range(_HEADS):
        sl = slice(h * _H, (h + 1) * _H)
        q0h = q0[:, sl]
        d0 = jnp.sum(q0h * k0[:, sl], axis=1, keepdims=True) * _SCALE
        d1 = jnp.sum(q0h * k1[:, sl], axis=1, keepdims=True) * _SCALE
        m = jnp.maximum(d0, d1)
        e0 = jnp.exp(d0 - m)
        e1 = jnp.exp(d1 - m)
        den = e0 + e1
        o_parts.append((e0 / den) * v0[:, sl] + (e1 / den) * v1[:, sl])
    o = jnp.concatenate(o_parts, axis=1)

    y = jax.lax.dot(o, Wout_ref[...], preferred_element_type=f32) + bout_ref[...]
    out_ref[...] = (jax.lax.dot(y, Wcls_ref[...], preferred_element_type=f32)
                    + bcls_ref[...])


@functools.partial(jax.jit, static_argnames=())
def _run(news_title, news_content, W_nt_t, b_nt_t, W_nt_c, b_nt_c,
         token, pos_emb, W_qkv, W_out, b_out, W_cls, b_cls):
    T = _TILE
    grid = (_B // T,)
    row_spec = pl.BlockSpec((T, 768), lambda i: (i, 0))

    def rep(shape):
        return pl.BlockSpec(shape, lambda i: tuple(0 for _ in shape))

    return pl.pallas_call(
        _fused_kernel,
        grid=grid,
        in_specs=[
            row_spec, row_spec,
            rep((768, 64)), rep((1, 64)),
            rep((768, 64)), rep((1, 64)),
            rep((1, 128)), rep((2, 128)),
            rep((128, 1536)),
            rep((512, 128)), rep((1, 128)),
            rep((128, 2)), rep((1, 2)),
        ],
        out_specs=pl.BlockSpec((T, 2), lambda i: (i, 0)),
        out_shape=jax.ShapeDtypeStruct((_B, 2), jnp.float32),
    )(news_title, news_content,
      W_nt_t, b_nt_t.reshape(1, 64), W_nt_c, b_nt_c.reshape(1, 64),
      token.reshape(1, 128), pos_emb.reshape(2, 128), W_qkv,
      W_out, b_out.reshape(1, 128), W_cls, b_cls.reshape(1, 2))


def kernel(news_title, news_content, tweet_content, tweet_profile, user_profile, user_description, source_description, W_nt_t, b_nt_t, W_nt_c, b_nt_c, W_tw_c, b_tw_c, W_tw_p, b_tw_p, W_us_p, b_us_p, W_us_d, b_us_d, W_sr_d, b_sr_d, W_rte_nt, b_rte_nt, W_rte_tu, b_rte_tu, W_rte_uu, b_rte_uu, W_rte_ns, b_rte_ns, token, pos_emb, W_qkv, W_out, b_out, W_cls, b_cls, t_news, t_tweet, t_user, t_source, ei_nt, ei_tu, ei_uu, ei_ns):
    return _run(news_title, news_content, W_nt_t, b_nt_t, W_nt_c, b_nt_c,
                token, pos_emb, W_qkv, W_out, b_out, W_cls, b_cls)
